# baseline (speedup 1.0000x reference)
import functools

import jax
import jax.numpy as jnp
from jax import lax
from jax.experimental import pallas as pl
from jax.experimental.pallas import tpu as pltpu
from jax.experimental.pallas import tpu_sc as plsc

B = 1000
NA = 50
D = 128
K = 12
NRBF = 50
LATENT = 128
HID = 256
NCLS = 20
MAXZ = 100
NBLK = 3
CUTOFF = 8.0

C = 20
NP = 56

SC_ROWS = 56064
SC_PER_W = SC_ROWS // 32
SC_CHUNK = SC_PER_W // 3


def _sc_embed(idx, table):
    mesh = plsc.VectorSubcoreMesh(core_axis_name="c", subcore_axis_name="s")

    @functools.partial(
        pl.kernel, mesh=mesh,
        out_type=jax.ShapeDtypeStruct((SC_ROWS, D), jnp.float32),
        scratch_types=[
            pltpu.VMEM((SC_CHUNK,), jnp.int32),
            pltpu.VMEM((SC_CHUNK, D), jnp.float32),
            pltpu.SemaphoreType.DMA,
        ],
    )
    def gather_kernel(idx_hbm, table_hbm, out_hbm, idx_v, rows_v, sem):
        wid = lax.axis_index("s") * 2 + lax.axis_index("c")
        for j in range(3):
            base = wid * SC_PER_W + j * SC_CHUNK
            pltpu.sync_copy(idx_hbm.at[pl.ds(base, SC_CHUNK)], idx_v)
            pltpu.async_copy(table_hbm.at[idx_v], rows_v, sem).wait()
            pltpu.sync_copy(rows_v, out_hbm.at[pl.ds(base, SC_CHUNK)])

    return gather_kernel(idx, table)


def _silu(x):
    return x / (1.0 + jnp.exp(-x))


def _mm(a, b):
    return jax.lax.dot_general(a, b, (((1,), (0,)), ((), ())),
                               preferred_element_type=jnp.float32,
                               precision=jax.lax.Precision.DEFAULT)


def _mm_onehot(a, b):
    bh = b.astype(jnp.bfloat16).astype(jnp.float32)
    bl = b - bh
    return _mm(a, bh) + _mm(a, bl)


def _bf(x):
    return x.astype(jnp.bfloat16).astype(jnp.float32)


def _fused_body(noise_ref, frow_ref, fcol_ref, h0_ref, len_ref, ang_ref,
                nw_ref, nb_ref, rbfw_ref, msgw_ref, updw_ref,
                updb_ref, outw_ref, outb_ref, f1w_ref, f1b_ref, f2w_ref,
                f2b_ref, f3w_ref, f3b_ref, out_ref):
    f32 = jnp.float32

    ln = len_ref[...].reshape(C, 3)
    ang = ang_ref[...].reshape(C, 3) * (jnp.pi / 180.0)
    cosv = jnp.cos(ang)
    sinv = jnp.sin(ang)

    def col(x, j):
        return x[:, j:j + 1].reshape(C, 1, 1)

    a = col(ln, 0); b = col(ln, 1); c = col(ln, 2)
    cos0 = col(cosv, 0); cos1 = col(cosv, 1); cos2 = col(cosv, 2)
    sin0 = col(sinv, 0); sin1 = col(sinv, 1)
    val = jnp.clip((cos0 * cos1 - cos2) / (sin0 * sin1), -1.0, 1.0)
    sin_gs = jnp.sqrt(jnp.maximum(1.0 - val * val, 0.0))
    c00 = a * sin1; c02 = a * cos1
    c10 = -b * sin0 * val; c11 = b * sin0 * sin_gs; c12 = b * cos0
    c22 = c

    frow = frow_ref[...]
    fcol = fcol_ref[...]

    def dcoord(k):
        dx = frow[:, :, k:k + 1] - fcol[:, k:k + 1, :]
        return dx - jnp.round(dx)

    dx0 = _bf(dcoord(0)); dx1 = _bf(dcoord(1)); dx2 = _bf(dcoord(2))
    b00 = _bf(c00); b02 = _bf(c02); b10 = _bf(c10)
    b11 = _bf(c11); b12 = _bf(c12); b22 = _bf(c22)
    cart0 = dx0 * b00 + dx1 * b10
    cart1 = dx1 * b11
    cart2 = dx0 * b02 + dx1 * b12 + dx2 * b22
    dist2 = cart0 * cart0 + cart1 * cart1 + cart2 * cart2

    ii = lax.broadcasted_iota(jnp.int32, (C, NP, NP), 1)
    jj = lax.broadcasted_iota(jnp.int32, (C, NP, NP), 2)
    dist2 = dist2 + jnp.where((ii == jj) | (jj >= NA), f32(1e9), f32(0.0))

    centers = (lax.broadcasted_iota(jnp.int32, (1, 1, NRBF), 2).astype(f32)
               * f32(CUTOFF / (NRBF - 1)))
    keys = ((lax.bitcast_convert_type(dist2, jnp.int32) & jnp.int32(-64))
            | jj)
    rbf_parts = []
    g_parts = []
    for _ in range(K):
        kmin = jnp.min(keys, axis=2, keepdims=True)
        amin = kmin & jnp.int32(63)
        m = lax.bitcast_convert_type(kmin & jnp.int32(-64), f32)
        dist = jnp.sqrt(m + f32(1e-12))
        rbf_parts.append(jnp.exp(-((dist - centers) ** 2) * f32(2.0)))
        g_parts.append((jj == amin).astype(f32))
        keys = jnp.where(keys == kmin, jnp.int32(0x7F000000), keys)
    rbf_all = jnp.concatenate(rbf_parts, axis=1)
    g_all = jnp.concatenate(g_parts, axis=1)
    rbf2 = rbf_all.reshape(C * NP * K, NRBF)

    nf = _silu(_bf(noise_ref[...].reshape(C, 1)) * _bf(nw_ref[...])
               + nb_ref[...])
    h = (h0_ref[...] + nf.reshape(C, 1, D)).reshape(C * NP, D)

    rbfw_cat = jnp.concatenate([rbfw_ref[0], rbfw_ref[1], rbfw_ref[2]],
                               axis=1)
    filt_all = _mm(rbf2, rbfw_cat)
    for l in range(NBLK):
        filt = filt_all[:, l * D:(l + 1) * D]
        hm = _mm(h, msgw_ref[l]).reshape(C, NP, D)
        gath = jnp.concatenate(
            [_mm(g_all[ci], hm[ci]) for ci in range(C)], axis=0)
        msg = gath * filt
        agg = jnp.sum(msg.reshape(C, K, NP, D), axis=1).reshape(C * NP, D)
        h = h + _silu(_mm(agg, updw_ref[l]) + updb_ref[l:l + 1, :])

    rmask = (lax.broadcasted_iota(jnp.int32, (C, NP, 1), 1) < NA).astype(f32)
    pooled = jnp.sum(h.reshape(C, NP, D) * rmask, axis=1) * f32(1.0 / NA)
    lat = _mm(pooled, outw_ref[...]) + outb_ref[...]
    x = jnp.maximum(_mm(lat, f1w_ref[...]) + f1b_ref[...], 0.0)
    x = jnp.maximum(_mm(x, f2w_ref[...]) + f2b_ref[...], 0.0)
    out_ref[...] = (_mm(x, f3w_ref[...]) + f3b_ref[...]).reshape(1, C, NCLS)


@functools.partial(jax.jit, static_argnames=("interpret",))
def _run(args, interpret=False):
    (noise2d, frow, fcol, h0, lengths, angles, noise_w,
     noise_b, rbf_w, msg_w, upd_w, upd_b, out_w, out_b, fc1_w, fc1_b,
     fc2_w, fc2_b, fc3_w, fc3_b) = args

    def blk(shape, imap):
        return pl.BlockSpec(shape, imap)

    def full(arr):
        r = arr.ndim
        return pl.BlockSpec(arr.shape, lambda i, _r=r: (0,) * _r)

    grid = (B // C,)
    in_specs = [
        blk((1, C, 1), lambda i: (i, 0, 0)),
        blk((C, NP, 3), lambda i: (i, 0, 0)),
        blk((C, 3, NP), lambda i: (i, 0, 0)),
        blk((C, NP, D), lambda i: (i, 0, 0)),
        blk((1, C, 3), lambda i: (i, 0, 0)),
        blk((1, C, 3), lambda i: (i, 0, 0)),
        full(noise_w), full(noise_b), full(rbf_w),
        full(msg_w), full(upd_w), full(upd_b), full(out_w), full(out_b),
        full(fc1_w), full(fc1_b), full(fc2_w), full(fc2_b), full(fc3_w),
        full(fc3_b),
    ]
    out = pl.pallas_call(
        _fused_body,
        grid=grid,
        in_specs=in_specs,
        out_specs=pl.BlockSpec((1, C, NCLS), lambda i: (i, 0, 0)),
        out_shape=jax.ShapeDtypeStruct((B // C, C, NCLS), jnp.float32),
        interpret=interpret,
    )(*args)
    return out.reshape(B, NCLS)


def kernel(noise_levels, frac_coords, atom_types, num_atoms, lengths,
           angles, atom_emb, noise_w, noise_b, rbf_w, msg_w, upd_w, upd_b,
           out_w, out_b, fc1_w, fc1_b, fc2_w, fc2_b, fc3_w, fc3_b,
           interpret=False):
    del num_atoms
    fc3d = jnp.pad(frac_coords.reshape(B, NA, 3), ((0, 0), (0, NP - NA), (0, 0)))
    types_pad = jnp.pad(atom_types.reshape(B, NA), ((0, 0), (0, NP - NA)))
    idx_flat = jnp.pad(types_pad.reshape(B * NP), (0, SC_ROWS - B * NP))
    h0 = _sc_embed(idx_flat, atom_emb)[:B * NP].reshape(B, NP, D)
    args = (
        noise_levels.reshape(B // C, C, 1),
        fc3d,
        jnp.transpose(fc3d, (0, 2, 1)),
        h0,
        lengths.reshape(B // C, C, 3),
        angles.reshape(B // C, C, 3),
        noise_w.reshape(1, D),
        noise_b.reshape(1, D),
        rbf_w,
        msg_w,
        upd_w,
        upd_b,
        out_w,
        out_b.reshape(1, LATENT),
        fc1_w,
        fc1_b.reshape(1, HID),
        fc2_w,
        fc2_b.reshape(1, HID),
        fc3_w,
        fc3_b.reshape(1, NCLS),
    )
    return _run(args, interpret=interpret)


# speedup vs baseline: 1.3264x; 1.3264x over previous
import functools

import jax
import jax.numpy as jnp
from jax import lax
from jax.experimental import pallas as pl

B = 1000
NA = 50
D = 128
K = 12
NRBF = 50
LATENT = 128
HID = 256
NCLS = 20
MAXZ = 100
NBLK = 3
CUTOFF = 8.0

C = 20
NP = 56


def _silu(x):
    return x / (1.0 + jnp.exp(-x))


def _mm(a, b):
    return jax.lax.dot_general(a, b, (((1,), (0,)), ((), ())),
                               preferred_element_type=jnp.float32,
                               precision=jax.lax.Precision.DEFAULT)


def _mm_onehot(a, b):
    bh = b.astype(jnp.bfloat16).astype(jnp.float32)
    bl = b - bh
    return _mm(a, bh) + _mm(a, bl)


def _bf(x):
    return x.astype(jnp.bfloat16).astype(jnp.float32)


def _fused_body(noise_ref, frow_ref, fcol_ref, types_ref, len_ref, ang_ref,
                emb_ref, nw_ref, nb_ref, rbfw_ref, msgw_ref, updw_ref,
                updb_ref, outw_ref, outb_ref, f1w_ref, f1b_ref, f2w_ref,
                f2b_ref, f3w_ref, f3b_ref, out_ref):
    f32 = jnp.float32

    ln = len_ref[...].reshape(C, 3)
    ang = ang_ref[...].reshape(C, 3) * (jnp.pi / 180.0)
    cosv = jnp.cos(ang)
    sinv = jnp.sin(ang)

    def col(x, j):
        return x[:, j:j + 1].reshape(C, 1, 1)

    a = col(ln, 0); b = col(ln, 1); c = col(ln, 2)
    cos0 = col(cosv, 0); cos1 = col(cosv, 1); cos2 = col(cosv, 2)
    sin0 = col(sinv, 0); sin1 = col(sinv, 1)
    val = jnp.clip((cos0 * cos1 - cos2) / (sin0 * sin1), -1.0, 1.0)
    sin_gs = jnp.sqrt(jnp.maximum(1.0 - val * val, 0.0))
    c00 = a * sin1; c02 = a * cos1
    c10 = -b * sin0 * val; c11 = b * sin0 * sin_gs; c12 = b * cos0
    c22 = c

    frow = frow_ref[...]
    fcol = fcol_ref[...]

    def dcoord(k):
        dx = frow[:, :, k:k + 1] - fcol[:, k:k + 1, :]
        return dx - jnp.round(dx)

    dx0 = _bf(dcoord(0)); dx1 = _bf(dcoord(1)); dx2 = _bf(dcoord(2))
    b00 = _bf(c00); b02 = _bf(c02); b10 = _bf(c10)
    b11 = _bf(c11); b12 = _bf(c12); b22 = _bf(c22)
    cart0 = dx0 * b00 + dx1 * b10
    cart1 = dx1 * b11
    cart2 = dx0 * b02 + dx1 * b12 + dx2 * b22
    dist2 = cart0 * cart0 + cart1 * cart1 + cart2 * cart2

    ii = lax.broadcasted_iota(jnp.int32, (C, NP, NP), 1)
    jj = lax.broadcasted_iota(jnp.int32, (C, NP, NP), 2)
    dist2 = dist2 + jnp.where((ii == jj) | (jj >= NA), f32(1e9), f32(0.0))

    centers = (lax.broadcasted_iota(jnp.int32, (1, 1, NRBF), 2).astype(f32)
               * f32(CUTOFF / (NRBF - 1)))
    keys = ((lax.bitcast_convert_type(dist2, jnp.int32) & jnp.int32(-64))
            | jj)
    rbf_parts = []
    g_parts = []
    for _ in range(K):
        kmin = jnp.min(keys, axis=2, keepdims=True)
        amin = kmin & jnp.int32(63)
        m = lax.bitcast_convert_type(kmin & jnp.int32(-64), f32)
        dist = jnp.sqrt(m + f32(1e-12))
        rbf_parts.append(jnp.exp(-((dist - centers) ** 2) * f32(2.0)))
        g_parts.append((jj == amin).astype(f32))
        keys = jnp.where(keys == kmin, jnp.int32(0x7F000000), keys)
    rbf_all = jnp.concatenate(rbf_parts, axis=1)
    g_all = jnp.concatenate(g_parts, axis=1)
    rbf2 = rbf_all.reshape(C * NP * K, NRBF)

    types = types_ref[...]
    zi = lax.broadcasted_iota(jnp.int32, (C, NP, MAXZ), 2)
    oh = (zi == types).astype(f32).reshape(C * NP, MAXZ)
    h = _mm_onehot(oh, emb_ref[...])
    nf = _silu(_bf(noise_ref[...].reshape(C, 1)) * _bf(nw_ref[...])
               + nb_ref[...])
    h = (h.reshape(C, NP, D) + nf.reshape(C, 1, D)).reshape(C * NP, D)

    rbfw_cat = jnp.concatenate([rbfw_ref[0], rbfw_ref[1], rbfw_ref[2]],
                               axis=1)
    filt_all = _mm(rbf2, rbfw_cat)
    for l in range(NBLK):
        filt = filt_all[:, l * D:(l + 1) * D]
        hm = _mm(h, msgw_ref[l]).reshape(C, NP, D)
        gath = jnp.concatenate(
            [_mm(g_all[ci], hm[ci]) for ci in range(C)], axis=0)
        msg = gath * filt
        agg = jnp.sum(msg.reshape(C, K, NP, D), axis=1).reshape(C * NP, D)
        h = h + _silu(_mm(agg, updw_ref[l]) + updb_ref[l:l + 1, :])

    rmask = (lax.broadcasted_iota(jnp.int32, (C, NP, 1), 1) < NA).astype(f32)
    pooled = jnp.sum(h.reshape(C, NP, D) * rmask, axis=1) * f32(1.0 / NA)
    lat = _mm(pooled, outw_ref[...]) + outb_ref[...]
    x = jnp.maximum(_mm(lat, f1w_ref[...]) + f1b_ref[...], 0.0)
    x = jnp.maximum(_mm(x, f2w_ref[...]) + f2b_ref[...], 0.0)
    out_ref[...] = (_mm(x, f3w_ref[...]) + f3b_ref[...]).reshape(1, C, NCLS)


@functools.partial(jax.jit, static_argnames=("interpret",))
def _run(args, interpret=False):
    (noise2d, frow, fcol, types3d, lengths, angles, atom_emb, noise_w,
     noise_b, rbf_w, msg_w, upd_w, upd_b, out_w, out_b, fc1_w, fc1_b,
     fc2_w, fc2_b, fc3_w, fc3_b) = args

    def blk(shape, imap):
        return pl.BlockSpec(shape, imap)

    def full(arr):
        r = arr.ndim
        return pl.BlockSpec(arr.shape, lambda i, _r=r: (0,) * _r)

    grid = (B // C,)
    in_specs = [
        blk((1, C, 1), lambda i: (i, 0, 0)),
        blk((C, NP, 3), lambda i: (i, 0, 0)),
        blk((C, 3, NP), lambda i: (i, 0, 0)),
        blk((C, NP, 1), lambda i: (i, 0, 0)),
        blk((1, C, 3), lambda i: (i, 0, 0)),
        blk((1, C, 3), lambda i: (i, 0, 0)),
        full(atom_emb), full(noise_w), full(noise_b), full(rbf_w),
        full(msg_w), full(upd_w), full(upd_b), full(out_w), full(out_b),
        full(fc1_w), full(fc1_b), full(fc2_w), full(fc2_b), full(fc3_w),
        full(fc3_b),
    ]
    out = pl.pallas_call(
        _fused_body,
        grid=grid,
        in_specs=in_specs,
        out_specs=pl.BlockSpec((1, C, NCLS), lambda i: (i, 0, 0)),
        out_shape=jax.ShapeDtypeStruct((B // C, C, NCLS), jnp.float32),
        interpret=interpret,
    )(*args)
    return out.reshape(B, NCLS)


def kernel(noise_levels, frac_coords, atom_types, num_atoms, lengths,
           angles, atom_emb, noise_w, noise_b, rbf_w, msg_w, upd_w, upd_b,
           out_w, out_b, fc1_w, fc1_b, fc2_w, fc2_b, fc3_w, fc3_b,
           interpret=False):
    del num_atoms
    fc3d = jnp.pad(frac_coords.reshape(B, NA, 3), ((0, 0), (0, NP - NA), (0, 0)))
    args = (
        noise_levels.reshape(B // C, C, 1),
        fc3d,
        jnp.transpose(fc3d, (0, 2, 1)),
        jnp.pad(atom_types.reshape(B, NA, 1), ((0, 0), (0, NP - NA), (0, 0))),
        lengths.reshape(B // C, C, 3),
        angles.reshape(B // C, C, 3),
        atom_emb,
        noise_w.reshape(1, D),
        noise_b.reshape(1, D),
        rbf_w,
        msg_w,
        upd_w,
        upd_b,
        out_w,
        out_b.reshape(1, LATENT),
        fc1_w,
        fc1_b.reshape(1, HID),
        fc2_w,
        fc2_b.reshape(1, HID),
        fc3_w,
        fc3_b.reshape(1, NCLS),
    )
    return _run(args, interpret=interpret)
